# Initial kernel scaffold; baseline (speedup 1.0000x reference)
import jax
import jax.numpy as jnp
from jax.experimental import pallas as pl


def kernel(x1, edge_index1, edge_weight1, x2, edge_index2, edge_weight2, W1a, b1a, W1b, b1b, W2a, b2a, W2b, b2b, fcW, fcb):
    raise NotImplementedError("write your pallas kernel here")


# speedup vs baseline: 7.9660x; 7.9660x over previous
import functools

import jax
import jax.numpy as jnp
from jax import lax
from jax.experimental import pallas as pl
from jax.experimental.pallas import tpu as pltpu
from jax.experimental.pallas import tpu_sc as plsc

N = 10000
E = 160000
D = 256
H = 128
TECS = 16
EPT = E // TECS
CH = 125
NCH = EPT // CH
ROWS2D = E // CH
NPAD = 10240
NPT = NPAD // TECS
DPT = NPAD // TECS

_mesh = plsc.VectorSubcoreMesh(core_axis_name="c", subcore_axis_name="s")

_f32 = jnp.float32
_i32 = jnp.int32


def _deg_body(dst1, ew1, dst2, ew2, out, dst_v, ew_v, zbuf, acc):
    c = lax.axis_index("c")
    s = lax.axis_index("s")

    @pl.when(c == 0)
    def _():
        pltpu.sync_copy(dst1.at[pl.ds(s * NCH, NCH)], dst_v)
        pltpu.sync_copy(ew1.at[pl.ds(s * NCH, NCH)], ew_v)

    @pl.when(c == 1)
    def _():
        pltpu.sync_copy(dst2.at[pl.ds(s * NCH, NCH)], dst_v)
        pltpu.sync_copy(ew2.at[pl.ds(s * NCH, NCH)], ew_v)

    def _zero(i, _):
        zbuf[pl.ds(i * 16, 16)] = jnp.zeros((16,), _f32)
        return 0

    lax.fori_loop(0, DPT // 16, _zero, 0)
    pltpu.sync_copy(zbuf, acc.at[pl.ds(s * DPT, DPT)])
    plsc.subcore_barrier()

    def _chunk(ci, _):
        pltpu.sync_copy(ew_v.at[ci], acc.at[dst_v.at[ci]], add=True)
        return 0

    lax.fori_loop(0, NCH, _chunk, 0)
    plsc.subcore_barrier()

    @pl.when(c == 0)
    def _():
        pltpu.sync_copy(acc.at[pl.ds(s * DPT, DPT)], out.at[0, pl.ds(s * DPT, DPT)])

    @pl.when(c == 1)
    def _():
        pltpu.sync_copy(acc.at[pl.ds(s * DPT, DPT)], out.at[1, pl.ds(s * DPT, DPT)])


_deg_call = functools.partial(
    pl.kernel,
    out_type=jax.ShapeDtypeStruct((2, NPAD), _f32),
    mesh=_mesh,
    scratch_types=[
        pltpu.VMEM((NCH, CH), _i32),
        pltpu.VMEM((NCH, CH), _f32),
        pltpu.VMEM((DPT,), _f32),
        pltpu.VMEM_SHARED((NPAD,), _f32),
    ],
)(_deg_body)


def _seg_body(src2d, dst2d, ewflat, g0, g1, out0, out1,
              src_v, dst_v, ew_v, rows_v, acc, sem):
    c = lax.axis_index("c")
    s = lax.axis_index("s")

    pltpu.sync_copy(src2d.at[pl.ds(s * NCH, NCH)], src_v)
    pltpu.sync_copy(dst2d.at[pl.ds(s * NCH, NCH)], dst_v)
    pltpu.sync_copy(ewflat.at[pl.ds(s * EPT, EPT)], ew_v.at[pl.ds(0, EPT)])

    def _zrow(r, _):
        for j in range(H // 16):
            rows_v[r, pl.ds(j * 16, 16)] = jnp.zeros((16,), _f32)
        return 0

    lax.fori_loop(0, CH, _zrow, 0)
    off = 0
    for sz in (120, 120, 120, 120, 120, 40):
        pltpu.sync_copy(rows_v.at[pl.ds(0, sz)],
                        acc.at[pl.ds(s * NPT + off, sz)])
        off += sz
    plsc.subcore_barrier()

    def _chunk(ci, _):
        @pl.when(c == 0)
        def _():
            pltpu.async_copy(g0.at[src_v.at[ci]], rows_v, sem).wait()

        @pl.when(c == 1)
        def _():
            pltpu.async_copy(g1.at[src_v.at[ci]], rows_v, sem).wait()

        def _srow(r, _):
            w = ew_v[pl.ds(ci * CH + r, 16)][0]
            for j in range(H // 16):
                rows_v[r, pl.ds(j * 16, 16)] = rows_v[r, pl.ds(j * 16, 16)] * w
            return 0

        lax.fori_loop(0, CH, _srow, 0)
        pltpu.sync_copy(rows_v, acc.at[dst_v.at[ci]], add=True)
        return 0

    lax.fori_loop(0, NCH, _chunk, 0)
    plsc.subcore_barrier()

    @pl.when(c == 0)
    def _():
        pltpu.sync_copy(acc.at[pl.ds(s * NPT, NPT)], out0.at[pl.ds(s * NPT, NPT)])

    @pl.when(c == 1)
    def _():
        pltpu.sync_copy(acc.at[pl.ds(s * NPT, NPT)], out1.at[pl.ds(s * NPT, NPT)])


_seg_call = functools.partial(
    pl.kernel,
    out_type=[jax.ShapeDtypeStruct((NPAD, H), _f32),
              jax.ShapeDtypeStruct((NPAD, H), _f32)],
    mesh=_mesh,
    scratch_types=[
        pltpu.VMEM((NCH, CH), _i32),
        pltpu.VMEM((NCH, CH), _i32),
        pltpu.VMEM((EPT + 16,), _f32),
        pltpu.VMEM((CH, H), _f32),
        pltpu.VMEM_SHARED((NPAD, H), _f32),
        pltpu.SemaphoreType.DMA,
    ],
)(_seg_body)


_BLK = 1000
_GRID = N // _BLK
_dims = (((1,), (1,)), ((), ()))


def _mm_a_body(x_ref, w_ref, deg_ref, g0_ref, g1_ref, dis_ref):
    d = deg_ref[...] + 1.0
    dis = jnp.where(d > 0, lax.rsqrt(d), 0.0)
    h = lax.dot_general(x_ref[...], w_ref[...], _dims,
                        preferred_element_type=_f32)
    g = h * dis
    g0_ref[...] = g[:, :H]
    g1_ref[...] = g[:, H:]
    dis_ref[...] = dis


def _mm_a(x, w, degcol):
    return pl.pallas_call(
        _mm_a_body,
        grid=(_GRID,),
        in_specs=[
            pl.BlockSpec((_BLK, D), lambda i: (i, 0)),
            pl.BlockSpec((D, D), lambda i: (0, 0)),
            pl.BlockSpec((_BLK, 1), lambda i: (i, 0)),
        ],
        out_specs=[
            pl.BlockSpec((_BLK, H), lambda i: (i, 0)),
            pl.BlockSpec((_BLK, H), lambda i: (i, 0)),
            pl.BlockSpec((_BLK, 1), lambda i: (i, 0)),
        ],
        out_shape=[
            jax.ShapeDtypeStruct((N, H), _f32),
            jax.ShapeDtypeStruct((N, H), _f32),
            jax.ShapeDtypeStruct((N, 1), _f32),
        ],
    )(x, w, degcol)


def _mm_b_body(s0_ref, s1_ref, g0_ref, g1_ref, dis_ref, b_ref, w_ref,
               o0_ref, o1_ref):
    dis = dis_ref[...]
    sv = jnp.concatenate([s0_ref[...], s1_ref[...]], axis=1)
    gv = jnp.concatenate([g0_ref[...], g1_ref[...]], axis=1)
    hin = jnp.maximum(dis * (sv + gv) + b_ref[...], 0.0)
    g = lax.dot_general(hin, w_ref[...], _dims,
                        preferred_element_type=_f32) * dis
    o0_ref[...] = g[:, :H]
    o1_ref[...] = g[:, H:]


def _mm_b(s0, s1, g0, g1, dis, b2d, w):
    return pl.pallas_call(
        _mm_b_body,
        grid=(_GRID,),
        in_specs=[
            pl.BlockSpec((_BLK, H), lambda i: (i, 0)),
            pl.BlockSpec((_BLK, H), lambda i: (i, 0)),
            pl.BlockSpec((_BLK, H), lambda i: (i, 0)),
            pl.BlockSpec((_BLK, H), lambda i: (i, 0)),
            pl.BlockSpec((_BLK, 1), lambda i: (i, 0)),
            pl.BlockSpec((1, D), lambda i: (0, 0)),
            pl.BlockSpec((D, D), lambda i: (0, 0)),
        ],
        out_specs=[
            pl.BlockSpec((_BLK, H), lambda i: (i, 0)),
            pl.BlockSpec((_BLK, H), lambda i: (i, 0)),
        ],
        out_shape=[
            jax.ShapeDtypeStruct((N, H), _f32),
            jax.ShapeDtypeStruct((N, H), _f32),
        ],
    )(s0, s1, g0, g1, dis, b2d, w)


def _fin_body(s10, s11, g10, g11, dis1, b1,
              s20, s21, g20, g21, dis2, b2,
              fca, fcb_w, fcb_b, out_ref, acc):
    i = pl.program_id(0)

    h1 = jnp.maximum(
        dis1[...] * (jnp.concatenate([s10[...], s11[...]], axis=1)
                     + jnp.concatenate([g10[...], g11[...]], axis=1))
        + b1[...], 0.0)
    h2 = jnp.maximum(
        dis2[...] * (jnp.concatenate([s20[...], s21[...]], axis=1)
                     + jnp.concatenate([g20[...], g21[...]], axis=1))
        + b2[...], 0.0)
    c1 = jnp.sum(h1, axis=0, keepdims=True)
    c2 = jnp.sum(h2, axis=0, keepdims=True)

    @pl.when(i == 0)
    def _():
        acc[0:1, :] = c1
        acc[1:2, :] = c2

    @pl.when(i > 0)
    def _():
        acc[0:1, :] = acc[0:1, :] + c1
        acc[1:2, :] = acc[1:2, :] + c2

    @pl.when(i == _GRID - 1)
    def _():
        m1 = acc[0:1, :] * (1.0 / N)
        m2 = acc[1:2, :] * (1.0 / N)
        z = (jnp.sum(m1 * fca[...]) + jnp.sum(m2 * fcb_w[...])
             + fcb_b[0, 0])
        out_ref[...] = jax.nn.sigmoid(z) * jnp.ones((1, 1), _f32)


def _final(s10, s11, g10, g11, dis1, b1,
           s20, s21, g20, g21, dis2, b2, fca, fcbw, fcbb):
    blk = [
        pl.BlockSpec((_BLK, H), lambda i: (i, 0)),
        pl.BlockSpec((_BLK, H), lambda i: (i, 0)),
        pl.BlockSpec((_BLK, H), lambda i: (i, 0)),
        pl.BlockSpec((_BLK, H), lambda i: (i, 0)),
        pl.BlockSpec((_BLK, 1), lambda i: (i, 0)),
        pl.BlockSpec((1, D), lambda i: (0, 0)),
    ]
    return pl.pallas_call(
        _fin_body,
        grid=(_GRID,),
        in_specs=blk + blk + [
            pl.BlockSpec((1, D), lambda i: (0, 0)),
            pl.BlockSpec((1, D), lambda i: (0, 0)),
            pl.BlockSpec((1, 1), lambda i: (0, 0)),
        ],
        out_specs=pl.BlockSpec((1, 1), lambda i: (0, 0)),
        out_shape=jax.ShapeDtypeStruct((1, 1), _f32),
        scratch_shapes=[pltpu.VMEM((8, D), _f32)],
    )(s10, s11, g10, g11, dis1, b1,
      s20, s21, g20, g21, dis2, b2, fca, fcbw, fcbb)


def _tower_pre(edge_index, edge_weight):
    src2d = edge_index[0].reshape(ROWS2D, CH)
    dst2d = edge_index[1].reshape(ROWS2D, CH)
    ew2d = edge_weight.reshape(ROWS2D, CH)
    return src2d, dst2d, ew2d


def kernel(x1, edge_index1, edge_weight1, x2, edge_index2, edge_weight2,
           W1a, b1a, W1b, b1b, W2a, b2a, W2b, b2b, fcW, fcb):
    src1, dst1, ew1 = _tower_pre(edge_index1, edge_weight1)
    src2, dst2, ew2 = _tower_pre(edge_index2, edge_weight2)

    degs = _deg_call(dst1, ew1, dst2, ew2)
    deg1 = degs[0, :N].reshape(N, 1)
    deg2 = degs[1, :N].reshape(N, 1)

    g1a0, g1a1, dis1 = _mm_a(x1, W1a, deg1)
    g2a0, g2a1, dis2 = _mm_a(x2, W2a, deg2)

    s1a0, s1a1 = [s[:N] for s in _seg_call(src1, dst1, edge_weight1, g1a0, g1a1)]
    s2a0, s2a1 = [s[:N] for s in _seg_call(src2, dst2, edge_weight2, g2a0, g2a1)]

    g1b0, g1b1 = _mm_b(s1a0, s1a1, g1a0, g1a1, dis1, b1a.reshape(1, D), W1b)
    g2b0, g2b1 = _mm_b(s2a0, s2a1, g2a0, g2a1, dis2, b2a.reshape(1, D), W2b)

    s1b0, s1b1 = [s[:N] for s in _seg_call(src1, dst1, edge_weight1, g1b0, g1b1)]
    s2b0, s2b1 = [s[:N] for s in _seg_call(src2, dst2, edge_weight2, g2b0, g2b1)]

    return _final(
        s1b0, s1b1, g1b0, g1b1, dis1, b1b.reshape(1, D),
        s2b0, s2b1, g2b0, g2b1, dis2, b2b.reshape(1, D),
        fcW[:, :D], fcW[:, D:], fcb.reshape(1, 1))
